# baseline (speedup 1.0000x reference)
import functools

import jax
import jax.numpy as jnp
from jax import lax
from jax.experimental import pallas as pl
from jax.experimental.pallas import tpu as pltpu
from jax.experimental.pallas import tpu_sc as plsc

N_CLS = 4096
N_CTX = 4
CTX_DIM = 512
CTX_LEN = 77
N_SUF = CTX_LEN - N_CTX - 1
VOCAB = 49408
IDX_STRIDE = 80

S = 3
A = 2

_info = plsc.get_sparse_core_info()
_NC = _info.num_cores
_NS = _info.num_subcores
_NW = _NC * _NS
_U = N_CLS // _NW


def _make_sc_call():
  mesh = plsc.VectorSubcoreMesh(core_axis_name="c", subcore_axis_name="s")

  @functools.partial(
      pl.kernel,
      mesh=mesh,
      compiler_params=pltpu.CompilerParams(use_tc_tiling_on_sc=False),
      out_type=jax.ShapeDtypeStruct((N_CLS, CTX_LEN, CTX_DIM), jnp.float32),
      scratch_types=[pltpu.VMEM((CTX_LEN,), jnp.int32) for _ in range(S)]
      + [pltpu.VMEM((CTX_LEN, CTX_DIM), jnp.float32) for _ in range(S)]
      + [pltpu.SemaphoreType.DMA] * (3 * S),
  )
  def sc_kernel(idx_hbm, table_hbm, out_hbm, *rest):
    idxs = rest[:S]
    rows = rest[S:2 * S]
    isem = rest[2 * S:3 * S]
    gsem = rest[3 * S:4 * S]
    wsem = rest[4 * S:5 * S]
    wid = lax.axis_index("s") * _NC + lax.axis_index("c")
    base = wid * _U

    def i_copy(u, s):
      return pltpu.make_async_copy(
          idx_hbm.at[pl.ds((base + u) * IDX_STRIDE, CTX_LEN)], idxs[s],
          isem[s])

    def g_copy(s):
      return pltpu.make_async_copy(table_hbm.at[idxs[s]], rows[s], gsem[s])

    def w_copy(u, s):
      return pltpu.make_async_copy(rows[s], out_hbm.at[base + u], wsem[s])

    def tick(u, k, drain_w, ahead, inext):
      sa = (k + A) % S
      if drain_w:
        w_copy(u - (S - A), sa).wait()
      if ahead:
        i_copy(u + A, sa).wait()
        g_copy(sa).start()
      g_copy(k).wait()
      w_copy(u, k).start()
      if inext:
        i_copy(u + S, k).start()

    for s in range(S):
      i_copy(s, s).start()
    for k in range(A):
      i_copy(k, k).wait()
      g_copy(k).start()

    front = S - A
    steady_len = ((_U - 2 * S + A) // S) * S
    steady_end = front + steady_len

    for u in range(front):
      tick(u, u % S, drain_w=False, ahead=True, inext=True)

    def body(gi, carry):
      u0 = front + gi * S
      for k2 in range(S):
        tick(u0 + k2, (front + k2) % S, drain_w=True, ahead=True, inext=True)
      return carry

    lax.fori_loop(0, steady_len // S, body, 0)

    for u in range(steady_end, _U):
      tick(u, u % S, drain_w=True, ahead=(u + A < _U), inext=(u + S < _U))
    for u in range(_U - (S - A), _U):
      w_copy(u, u % S).wait()

  return sc_kernel


_sc_call = _make_sc_call()


def kernel(tokenized_prompts, ctx, token_embedding):
  table_ext = jnp.concatenate([token_embedding, ctx], axis=0)
  ctx_ids = jnp.broadcast_to(
      jnp.arange(VOCAB, VOCAB + N_CTX, dtype=jnp.int32)[None, :],
      (N_CLS, N_CTX))
  idx = jnp.concatenate(
      [
          tokenized_prompts[:, :1],
          ctx_ids,
          tokenized_prompts[:, 1:1 + N_SUF],
          jnp.zeros((N_CLS, IDX_STRIDE - CTX_LEN), jnp.int32),
      ],
      axis=1,
  ).reshape(-1)
  prompts = _sc_call(idx, table_ext)
  return (prompts, tokenized_prompts)
